# baseline (speedup 1.0000x reference)
import jax
import jax.numpy as jnp
from jax import lax
from jax.experimental import pallas as pl
from jax.experimental.pallas import tpu as pltpu
from jax.experimental.pallas import tpu_sc as plsc

_TOKENS = 49408
_D = 512
_S = 77
_SP = 80
_B = 4096

_info = plsc.get_sparse_core_info()
_NC, _NS, _L = _info.num_cores, _info.num_subcores, _info.num_lanes
_NW = _NC * _NS
_BPW = _B // _NW
_IDG = 32


def _body(ids_hbm, tok_hbm, pos_hbm, out_hbm, ids_v, pos_v, buf_v, obuf_v,
          gsem, ssem):
    wid = lax.axis_index("s") * _NC + lax.axis_index("c")
    b0 = wid * _BPW
    pltpu.sync_copy(pos_hbm, pos_v)

    def group(g, _):
        gb = b0 + g * _IDG
        pltpu.sync_copy(ids_hbm.at[pl.ds(gb, _IDG)], ids_v)

        def chunk(c, _):
            pltpu.async_copy(tok_hbm.at[ids_v.at[c]], buf_v, gsem).wait()

            def row(i, _):
                def col(j, _):
                    sl = pl.ds(j * _L, _L)
                    obuf_v[i, sl] = buf_v[i, sl] + pos_v[i, sl]
                    return 0
                return lax.fori_loop(0, _D // _L, col, 0)

            lax.fori_loop(0, _S, row, 0)
            pltpu.async_copy(obuf_v, out_hbm.at[gb + c], ssem).wait()
            return 0

        lax.fori_loop(0, _IDG, chunk, 0)
        return 0

    lax.fori_loop(0, _BPW // _IDG, group, 0)


def kernel(input_ids, token_table, position_table):
    ids_p = jnp.pad(input_ids.astype(jnp.int32), ((0, 0), (0, _SP - _S)))
    mesh = plsc.VectorSubcoreMesh(core_axis_name="c", subcore_axis_name="s")
    f = pl.kernel(
        _body,
        out_type=jax.ShapeDtypeStruct((_B, _S, _D), jnp.float32),
        mesh=mesh,
        scratch_types=[
            pltpu.VMEM((_IDG, _SP), jnp.int32),
            pltpu.VMEM((_S, _D), jnp.float32),
            pltpu.VMEM((_SP, _D), jnp.float32),
            pltpu.VMEM((_S, _D), jnp.float32),
            pltpu.SemaphoreType.DMA,
            pltpu.SemaphoreType.DMA,
        ],
    )
    return f(ids_p, token_table, position_table)
